# Initial kernel scaffold; baseline (speedup 1.0000x reference)
import functools

import jax
import jax.numpy as jnp
from jax import lax
from jax.experimental import pallas as pl
from jax.experimental.pallas import tpu as pltpu
from jax.experimental.pallas import tpu_sc as plsc

_INFO = plsc.get_sparse_core_info()
_NC = _INFO.num_cores
_NS = _INFO.num_subcores
_NW = _NC * _NS


def _make_lookup(n, embed, chunk):
    assert n % (_NW * chunk) == 0
    per_w = n // _NW
    n_chunks = per_w // chunk
    mesh = plsc.VectorSubcoreMesh(core_axis_name="c", subcore_axis_name="s")

    @functools.partial(
        pl.kernel,
        out_type=jax.ShapeDtypeStruct((n, embed), jnp.float32),
        mesh=mesh,
        scratch_types=[
            pltpu.VMEM((chunk,), jnp.int32),
            pltpu.VMEM((chunk, embed), jnp.float32),
            pltpu.SemaphoreType.DMA,
        ],
    )
    def lookup(day_hbm, table_hbm, out_hbm, idx_v, rows_v, sem):
        wid = lax.axis_index("s") * _NC + lax.axis_index("c")
        w_base = wid * per_w

        def chunk_body(i, carry):
            base = w_base + i * chunk
            pltpu.sync_copy(day_hbm.at[pl.ds(base, chunk)], idx_v)
            pltpu.async_copy(table_hbm.at[idx_v], rows_v, sem).wait()
            pltpu.sync_copy(rows_v, out_hbm.at[pl.ds(base, chunk)])
            return carry

        lax.fori_loop(0, n_chunks, chunk_body, 0)

    return lookup


def kernel(day, table):
    batch, hist = day.shape
    vocab, embed = table.shape
    n = batch * hist
    day_flat = day.reshape(n).astype(jnp.int32)
    lookup = _make_lookup(n, embed, chunk=1024)
    out = lookup(day_flat, table)
    return out.reshape(batch, hist, embed)


# speedup vs baseline: 2.7697x; 2.7697x over previous
import functools

import jax
import jax.numpy as jnp
from jax import lax
from jax.experimental import pallas as pl
from jax.experimental.pallas import tpu as pltpu
from jax.experimental.pallas import tpu_sc as plsc

_INFO = plsc.get_sparse_core_info()
_NC = _INFO.num_cores
_NS = _INFO.num_subcores
_NW = _NC * _NS


def _make_lookup(n, embed, chunk):
    assert n % (_NW * chunk) == 0
    per_w = n // _NW
    n_chunks = per_w // chunk
    mesh = plsc.VectorSubcoreMesh(core_axis_name="c", subcore_axis_name="s")

    @functools.partial(
        pl.kernel,
        out_type=jax.ShapeDtypeStruct((n, embed), jnp.float32),
        mesh=mesh,
        scratch_types=[
            pltpu.VMEM((chunk,), jnp.int32),
            pltpu.VMEM((chunk, embed), jnp.float32),
            pltpu.SemaphoreType.DMA,
        ],
        compiler_params=pltpu.CompilerParams(use_tc_tiling_on_sc=False),
    )
    def lookup(day_hbm, table_hbm, out_hbm, idx_v, rows_v, sem):
        wid = lax.axis_index("s") * _NC + lax.axis_index("c")
        w_base = wid * per_w

        def chunk_body(i, carry):
            base = w_base + i * chunk
            pltpu.sync_copy(day_hbm.at[pl.ds(base, chunk)], idx_v)
            pltpu.async_copy(table_hbm.at[idx_v], rows_v, sem).wait()
            pltpu.sync_copy(rows_v, out_hbm.at[pl.ds(base, chunk)])
            return carry

        lax.fori_loop(0, n_chunks, chunk_body, 0)

    return lookup


def kernel(day, table):
    batch, hist = day.shape
    vocab, embed = table.shape
    n = batch * hist
    day_flat = day.reshape(n).astype(jnp.int32)
    lookup = _make_lookup(n, embed, chunk=1024)
    out = lookup(day_flat, table)
    return out.reshape(batch, hist, embed)


# speedup vs baseline: 5.7234x; 2.0664x over previous
import functools

import jax
import jax.numpy as jnp
from jax import lax
from jax.experimental import pallas as pl
from jax.experimental.pallas import tpu as pltpu
from jax.experimental.pallas import tpu_sc as plsc

_INFO = plsc.get_sparse_core_info()
_NC = _INFO.num_cores
_NS = _INFO.num_subcores
_NW = _NC * _NS


def _make_lookup(n, vocab, embed, chunk):
    assert n % (_NW * chunk) == 0
    per_w = n // _NW
    n_chunks = per_w // chunk
    mesh = plsc.VectorSubcoreMesh(core_axis_name="c", subcore_axis_name="s")

    @functools.partial(
        pl.kernel,
        out_type=jax.ShapeDtypeStruct((n, embed), jnp.float32),
        mesh=mesh,
        scratch_types=[
            pltpu.VMEM_SHARED((vocab, embed), jnp.float32),
            pltpu.VMEM((2, chunk), jnp.int32),
            pltpu.VMEM((2, chunk, embed), jnp.float32),
            pltpu.SemaphoreType.DMA,
            pltpu.SemaphoreType.DMA,
            pltpu.SemaphoreType.DMA,
            pltpu.SemaphoreType.DMA,
        ],
        compiler_params=pltpu.CompilerParams(use_tc_tiling_on_sc=False),
    )
    def lookup(day_hbm, table_hbm, out_hbm, table_v, idx_v, rows_v,
               g_sem0, g_sem1, o_sem0, o_sem1):
        wid = lax.axis_index("s") * _NC + lax.axis_index("c")
        w_base = wid * per_w
        g_sems = (g_sem0, g_sem1)
        o_sems = (o_sem0, o_sem1)

        pltpu.sync_copy(table_hbm, table_v)
        pltpu.sync_copy(day_hbm.at[pl.ds(w_base, chunk)], idx_v.at[0])
        pltpu.async_copy(table_v.at[idx_v.at[0]], rows_v.at[0], g_sems[0])
        pltpu.sync_copy(day_hbm.at[pl.ds(w_base + chunk, chunk)], idx_v.at[1])

        def chunk_body(g, carry):
            for b in (0, 1):

                @pl.when(g % 2 == b)
                def _():
                    nb = 1 - b
                    base = w_base + g * chunk

                    @pl.when(g >= 1)
                    def _():
                        pltpu.make_async_copy(
                            rows_v.at[nb], out_hbm.at[pl.ds(base, chunk)],
                            o_sems[nb]).wait()

                    @pl.when(g + 1 < n_chunks)
                    def _():
                        pltpu.async_copy(table_v.at[idx_v.at[nb]],
                                         rows_v.at[nb], g_sems[nb])

                    pltpu.make_async_copy(table_v.at[idx_v.at[b]],
                                          rows_v.at[b], g_sems[b]).wait()

                    @pl.when(g + 2 < n_chunks)
                    def _():
                        pltpu.sync_copy(
                            day_hbm.at[pl.ds(base + 2 * chunk, chunk)],
                            idx_v.at[b])

                    pltpu.async_copy(rows_v.at[b],
                                     out_hbm.at[pl.ds(base, chunk)],
                                     o_sems[b])

            return carry

        lax.fori_loop(0, n_chunks, chunk_body, 0)
        lb = (n_chunks - 1) % 2
        pltpu.make_async_copy(
            rows_v.at[lb],
            out_hbm.at[pl.ds(w_base + (n_chunks - 1) * chunk, chunk)],
            o_sems[lb]).wait()

    return lookup


def kernel(day, table):
    batch, hist = day.shape
    vocab, embed = table.shape
    n = batch * hist
    day_flat = day.reshape(n).astype(jnp.int32)
    lookup = _make_lookup(n, vocab, embed, chunk=800)
    out = lookup(day_flat, table)
    return out.reshape(batch, hist, embed)
